# baseline (speedup 1.0000x reference)
import functools

import jax
import jax.numpy as jnp
from jax import lax
from jax.experimental import pallas as pl
from jax.experimental.pallas import tpu as pltpu
from jax.experimental.pallas import tpu_sc as plsc

NC = 2
NS = 16
NW = NC * NS

BATCH = 16384
HIST = 50
D = 64
BPW = BATCH // NW
NBUF = 8
NIF = 5


def _tile_body(pe_hbm, idx_hbm, out_hbm, idx_v, rows_v, gsem):
  wid = lax.axis_index("s") * NC + lax.axis_index("c")
  base = wid * BPW

  pltpu.sync_copy(idx_hbm.at[pl.ds(base, BPW)], idx_v)

  def start(i):
    pltpu.async_copy(pe_hbm.at[idx_v.at[i]], rows_v.at[i % NBUF], gsem)

  def wait_gather(i):
    pltpu.make_async_copy(
        pe_hbm.at[idx_v.at[i]], rows_v.at[i % NBUF], gsem).wait()

  def drain(i):
    pltpu.sync_copy(rows_v.at[i % NBUF], out_hbm.at[base + i])

  for i in range(NIF):
    start(i)

  @pl.loop(0, BPW // NBUF - 1)
  def _(g):
    for b in range(NBUF):
      i = g * NBUF + b
      wait_gather(i)
      start(i + NIF)
      drain(i)

  for b in range(NBUF):
    i = BPW - NBUF + b
    wait_gather(i)
    if b < NBUF - NIF:
      start(i + NIF)
    drain(i)


@functools.partial(
    pl.kernel,
    out_type=jax.ShapeDtypeStruct((BATCH, HIST, D), jnp.float32),
    mesh=plsc.VectorSubcoreMesh(
        core_axis_name="c", subcore_axis_name="s",
        num_cores=NC, num_subcores=NS),
    scratch_types=[
        pltpu.VMEM((BPW, HIST), jnp.int32),
        pltpu.VMEM((NBUF, HIST, D), jnp.float32),
        pltpu.SemaphoreType.DMA,
    ],
    compiler_params=pltpu.CompilerParams(use_tc_tiling_on_sc=False),
)
def _gather(pe_hbm, idx_hbm, out_hbm, idx_v, rows_v, gsem):
  _tile_body(pe_hbm, idx_hbm, out_hbm, idx_v, rows_v, gsem)


def kernel(m, pe):
  return _gather(pe, m.astype(jnp.int32))


# speedup vs baseline: 1.3588x; 1.3588x over previous
import functools

import jax
import jax.numpy as jnp
from jax import lax
from jax.experimental import pallas as pl
from jax.experimental.pallas import tpu as pltpu
from jax.experimental.pallas import tpu_sc as plsc

NC = 2
NS = 16
NW = NC * NS

BATCH = 16384
HIST = 50
D = 64
DW = 128
BPW = BATCH // NW
NBUF = 6
NIF = 4


def _tile_body(pe_hbm, idx_hbm, out_hbm, idx_v, rows_v, gsem):
  wid = lax.axis_index("s") * NC + lax.axis_index("c")
  base = wid * BPW

  pltpu.sync_copy(idx_hbm.at[wid], idx_v)

  def start(i):
    pltpu.async_copy(
        pe_hbm.at[idx_v.at[i, pl.ds(0, HIST)]], rows_v.at[i % NBUF], gsem)

  def wait_gather(i):
    pltpu.make_async_copy(
        pe_hbm.at[idx_v.at[i, pl.ds(0, HIST)]], rows_v.at[i % NBUF],
        gsem).wait()

  def drain(i):
    pltpu.sync_copy(rows_v.at[i % NBUF], out_hbm.at[base + i])

  for i in range(NIF):
    start(i)

  @pl.loop(0, BPW // NBUF - 1)
  def _(g):
    for b in range(NBUF):
      i = g * NBUF + b
      wait_gather(i)
      start(i + NIF)
      drain(i)

  for b in range(NBUF):
    i = (BPW // NBUF - 1) * NBUF + b
    wait_gather(i)
    if i + NIF < BPW:
      start(i + NIF)
    drain(i)

  @pl.loop((BPW // NBUF) * NBUF, BPW)
  def _(i):
    wait_gather(i)
    drain(i)


@functools.partial(
    pl.kernel,
    out_type=jax.ShapeDtypeStruct((BATCH, HIST, DW), jnp.float32),
    mesh=plsc.VectorSubcoreMesh(
        core_axis_name="c", subcore_axis_name="s",
        num_cores=NC, num_subcores=NS),
    scratch_types=[
        pltpu.VMEM((BPW, DW), jnp.int32),
        pltpu.VMEM((NBUF, HIST, DW), jnp.float32),
        pltpu.SemaphoreType.DMA,
    ],
    compiler_params=pltpu.CompilerParams(use_tc_tiling_on_sc=True),
)
def _gather(pe_hbm, idx_hbm, out_hbm, idx_v, rows_v, gsem):
  _tile_body(pe_hbm, idx_hbm, out_hbm, idx_v, rows_v, gsem)


def kernel(m, pe):
  pe_wide = jnp.pad(pe, ((0, 0), (0, DW - D)))
  idx = jnp.pad(
      m.astype(jnp.int32).reshape(NW, BPW, HIST),
      ((0, 0), (0, 0), (0, DW - HIST)))
  return _gather(pe_wide, idx)[:, :, :D]


# speedup vs baseline: 1.3621x; 1.0025x over previous
import functools

import jax
import jax.numpy as jnp
from jax import lax
from jax.experimental import pallas as pl
from jax.experimental.pallas import tpu as pltpu
from jax.experimental.pallas import tpu_sc as plsc

NC = 2
NS = 16
NW = NC * NS

BATCH = 16384
HIST = 50
D = 64
DW = 128
BPW = BATCH // NW
NBUF = 8
NIF = 5


def _tile_body(pe_hbm, idx_hbm, out_hbm, idx_v, rows_v, gsem, osem):
  wid = lax.axis_index("s") * NC + lax.axis_index("c")
  base = wid * BPW

  pltpu.sync_copy(idx_hbm.at[wid], idx_v)

  def start(i):
    pltpu.async_copy(
        pe_hbm.at[idx_v.at[i, pl.ds(0, HIST)]], rows_v.at[i % NBUF], gsem)

  def wait_gather(i):
    pltpu.make_async_copy(
        pe_hbm.at[idx_v.at[i, pl.ds(0, HIST)]], rows_v.at[i % NBUF],
        gsem).wait()

  def drain(i):
    pltpu.async_copy(rows_v.at[i % NBUF], out_hbm.at[base + i], osem)

  def wait_drain(i):
    pltpu.make_async_copy(
        rows_v.at[i % NBUF], out_hbm.at[base + i], osem).wait()

  for i in range(NIF):
    start(i)

  for i in range(NBUF):
    wait_gather(i)
    if i + NIF - NBUF >= 0:
      wait_drain(i + NIF - NBUF)
    start(i + NIF)
    drain(i)

  @pl.loop(1, BPW // NBUF - 1)
  def _(g):
    for b in range(NBUF):
      i = g * NBUF + b
      wait_gather(i)
      wait_drain(i + NIF - NBUF)
      start(i + NIF)
      drain(i)

  for b in range(NBUF):
    i = (BPW // NBUF - 1) * NBUF + b
    wait_gather(i)
    if i + NIF < BPW:
      wait_drain(i + NIF - NBUF)
      start(i + NIF)
    drain(i)

  @pl.loop((BPW // NBUF) * NBUF, BPW)
  def _(i):
    wait_gather(i)
    drain(i)

  @pl.loop(BPW - NBUF, BPW)
  def _(i):
    wait_drain(i)


@functools.partial(
    pl.kernel,
    out_type=jax.ShapeDtypeStruct((BATCH, HIST, DW), jnp.float32),
    mesh=plsc.VectorSubcoreMesh(
        core_axis_name="c", subcore_axis_name="s",
        num_cores=NC, num_subcores=NS),
    scratch_types=[
        pltpu.VMEM((BPW, DW), jnp.int32),
        pltpu.VMEM((NBUF, HIST, DW), jnp.float32),
        pltpu.SemaphoreType.DMA,
        pltpu.SemaphoreType.DMA,
    ],
    compiler_params=pltpu.CompilerParams(use_tc_tiling_on_sc=True),
)
def _gather(pe_hbm, idx_hbm, out_hbm, idx_v, rows_v, gsem, osem):
  _tile_body(pe_hbm, idx_hbm, out_hbm, idx_v, rows_v, gsem, osem)


def kernel(m, pe):
  pe_wide = jnp.pad(pe, ((0, 0), (0, DW - D)))
  idx = jnp.pad(
      m.astype(jnp.int32).reshape(NW, BPW, HIST),
      ((0, 0), (0, 0), (0, DW - HIST)))
  return _gather(pe_wide, idx)[:, :, :D]
